# baseline (speedup 1.0000x reference)
import functools

import jax
import jax.numpy as jnp
from jax import lax
from jax.experimental import pallas as pl
from jax.experimental.pallas import tpu as pltpu
from jax.experimental.pallas import tpu_sc as plsc

VOCAB = 1000000
DIM = 64
BATCH = 16384
SEQ = 50
NC, NS = 2, 16
NW = NC * NS
BW = BATCH // NW
SUB = 128
CPB = BW // SUB
NCHUNK = SEQ * CPB
W2ROWS = VOCAB // 2

_mesh = plsc.VectorSubcoreMesh(core_axis_name="c", subcore_axis_name="s")


@functools.partial(
    pl.kernel,
    out_type=jax.ShapeDtypeStruct((SEQ * DIM, BATCH), jnp.float32),
    mesh=_mesh,
    scratch_types=[
        pltpu.VMEM((SEQ, BW), jnp.int32),
        pltpu.VMEM((SUB, 2 * DIM), jnp.float32),
        pltpu.VMEM((SUB, 2 * DIM), jnp.float32),
        pltpu.VMEM((DIM, SUB), jnp.float32),
        pltpu.VMEM((DIM, SUB), jnp.float32),
        pltpu.VMEM((SUB,), jnp.int32),
        pltpu.VMEM((SUB,), jnp.int32),
        pltpu.SemaphoreType.DMA,
        pltpu.SemaphoreType.DMA,
        pltpu.SemaphoreType.DMA,
        pltpu.SemaphoreType.DMA,
    ],
    compiler_params=pltpu.CompilerParams(
        use_tc_tiling_on_sc=True, needs_layout_passes=False,
        disable_bounds_checks=True),
)
def _emb_kernel(w2_hbm, xt_hbm, out_hbm, xw, g0, g1, o0, o1, i0, i1,
                gsem0, gsem1, wsem0, wsem1):
    wid = lax.axis_index("s") * NC + lax.axis_index("c")
    b0 = wid * BW
    pltpu.sync_copy(xt_hbm.at[:, pl.ds(b0, BW)], xw)

    lane = jax.lax.broadcasted_iota(jnp.int32, (16,), 0)

    def xvec(t, g):
        s = t // CPB
        col = (t % CPB) * SUB + g * 16
        row_idx = jnp.full((16,), s, jnp.int32)
        return plsc.load_gather(xw, [row_idx, lane + col])

    def compute_idx(t, ib):
        def body(g, carry):
            ib[pl.ds(g * 16, 16)] = xvec(t, g) >> 1
            return carry
        lax.fori_loop(0, 8, body, 0)

    def fire(gb, ib, gsem):
        pltpu.async_copy(w2_hbm.at[ib], gb, gsem)

    def drain(gb, gsem):
        pltpu.make_async_copy(w2_hbm.at[pl.ds(0, SUB)], gb, gsem).wait()

    def extract(t, gb, ob):
        BB = 16
        def body(g, carry):
            colb = lane + g * 16
            pv = (xvec(t, g) & 1) * DIM
            cv = lane
            vals, cvs = [], []
            for k in range(BB):
                vals.append(plsc.load_gather(gb, [colb, pv + cv]))
                cvs.append(cv)
                cv = (cv + 1) & (DIM - 1)
            for blk in range(1, DIM // BB):
                nvals, ncvs = [], []
                for k in range(BB):
                    nvals.append(plsc.load_gather(gb, [colb, pv + cv]))
                    ncvs.append(cv)
                    cv = (cv + 1) & (DIM - 1)
                for k in range(BB):
                    plsc.store_scatter(ob, [cvs[k], colb], vals[k])
                vals, cvs = nvals, ncvs
            for k in range(BB):
                plsc.store_scatter(ob, [cvs[k], colb], vals[k])
            return carry
        lax.fori_loop(0, 8, body, 0)

    def write(t, ob, wsem):
        s = t // CPB
        col = b0 + (t % CPB) * SUB
        return pltpu.async_copy(
            ob, out_hbm.at[pl.ds(s * DIM, DIM), pl.ds(col, SUB)], wsem)

    compute_idx(0, i0)
    fire(g0, i0, gsem0)
    compute_idx(1, i1)
    fire(g1, i1, gsem1)

    drain(g0, gsem0)
    compute_idx(2, i0)
    extract(0, g0, o0)
    fire(g0, i0, gsem0)
    write(0, o0, wsem0)
    drain(g1, gsem1)
    compute_idx(3, i1)
    extract(1, g1, o1)
    fire(g1, i1, gsem1)
    write(1, o1, wsem1)

    def pair_body(i, carry):
        a = 2 * i
        drain(g0, gsem0)
        compute_idx(a + 2, i0)
        pltpu.make_async_copy(o0, out_hbm.at[pl.ds(0, DIM), pl.ds(0, SUB)], wsem0).wait()
        extract(a, g0, o0)
        fire(g0, i0, gsem0)
        write(a, o0, wsem0)
        drain(g1, gsem1)
        compute_idx(a + 3, i1)
        pltpu.make_async_copy(o1, out_hbm.at[pl.ds(0, DIM), pl.ds(0, SUB)], wsem1).wait()
        extract(a + 1, g1, o1)
        fire(g1, i1, gsem1)
        write(a + 1, o1, wsem1)
        return carry

    lax.fori_loop(1, NCHUNK // 2 - 1, pair_body, 0)

    drain(g0, gsem0)
    pltpu.make_async_copy(o0, out_hbm.at[pl.ds(0, DIM), pl.ds(0, SUB)], wsem0).wait()
    extract(NCHUNK - 2, g0, o0)
    w0 = write(NCHUNK - 2, o0, wsem0)
    drain(g1, gsem1)
    pltpu.make_async_copy(o1, out_hbm.at[pl.ds(0, DIM), pl.ds(0, SUB)], wsem1).wait()
    extract(NCHUNK - 1, g1, o1)
    w1 = write(NCHUNK - 1, o1, wsem1)
    w0.wait()
    w1.wait()


def kernel(x, weight):
    w1 = lax.optimization_barrier(weight.reshape(VOCAB * DIM))
    w2 = w1.reshape(W2ROWS, 2 * DIM)
    xt = x.T.astype(jnp.int32)
    out = _emb_kernel(w2, xt)
    return jnp.transpose(out.reshape(SEQ, DIM, BATCH), (2, 0, 1))


# speedup vs baseline: 1.1302x; 1.1302x over previous
import functools

import jax
import jax.numpy as jnp
from jax import lax
from jax.experimental import pallas as pl
from jax.experimental.pallas import tpu as pltpu
from jax.experimental.pallas import tpu_sc as plsc

VOCAB = 1000000
DIM = 64
BATCH = 16384
SEQ = 50
NC, NS = 2, 16
NW = NC * NS
BW = BATCH // NW
SUB = 128
CPB = BW // SUB
NCHUNK = SEQ * CPB
W2ROWS = VOCAB

_mesh = plsc.VectorSubcoreMesh(core_axis_name="c", subcore_axis_name="s")


@functools.partial(
    pl.kernel,
    out_type=jax.ShapeDtypeStruct((SEQ * DIM, BATCH), jnp.float32),
    mesh=_mesh,
    scratch_types=[
        pltpu.VMEM((SEQ, BW), jnp.int32),
        pltpu.VMEM((SUB, 2 * DIM), jnp.float32),
        pltpu.VMEM((SUB, 2 * DIM), jnp.float32),
        pltpu.VMEM((DIM, SUB), jnp.float32),
        pltpu.VMEM((DIM, SUB), jnp.float32),
        pltpu.VMEM((SUB,), jnp.int32),
        pltpu.VMEM((SUB,), jnp.int32),
        pltpu.SemaphoreType.DMA,
        pltpu.SemaphoreType.DMA,
        pltpu.SemaphoreType.DMA,
        pltpu.SemaphoreType.DMA,
    ],
    compiler_params=pltpu.CompilerParams(
        use_tc_tiling_on_sc=True, needs_layout_passes=False,
        disable_bounds_checks=True),
)
def _emb_kernel(w2_hbm, xt_hbm, out_hbm, xw, g0, g1, o0, o1, i0, i1,
                gsem0, gsem1, wsem0, wsem1):
    wid = lax.axis_index("s") * NC + lax.axis_index("c")
    b0 = wid * BW
    pltpu.sync_copy(xt_hbm.at[:, pl.ds(b0, BW)], xw)

    lane = jax.lax.broadcasted_iota(jnp.int32, (16,), 0)

    def xvec(t, g):
        s = t // CPB
        col = (t % CPB) * SUB + g * 16
        row_idx = jnp.full((16,), s, jnp.int32)
        return plsc.load_gather(xw, [row_idx, lane + col])

    def compute_idx(t, ib):
        def body(g, carry):
            ib[pl.ds(g * 16, 16)] = xvec(t, g)
            return carry
        lax.fori_loop(0, 8, body, 0)

    def fire(gb, ib, gsem):
        pltpu.async_copy(w2_hbm.at[ib], gb, gsem)

    def drain(gb, gsem):
        pltpu.make_async_copy(w2_hbm.at[pl.ds(0, SUB)], gb, gsem).wait()

    def extract(t, gb, ob):
        BB = 16
        def body(g, carry):
            colb = lane + g * 16
            cv = lane
            vals, cvs = [], []
            for k in range(BB):
                vals.append(plsc.load_gather(gb, [colb, cv]))
                cvs.append(cv)
                cv = (cv + 1) & (DIM - 1)
            for blk in range(1, DIM // BB):
                nvals, ncvs = [], []
                for k in range(BB):
                    nvals.append(plsc.load_gather(gb, [colb, cv]))
                    ncvs.append(cv)
                    cv = (cv + 1) & (DIM - 1)
                for k in range(BB):
                    plsc.store_scatter(ob, [cvs[k], colb], vals[k])
                vals, cvs = nvals, ncvs
            for k in range(BB):
                plsc.store_scatter(ob, [cvs[k], colb], vals[k])
            return carry
        lax.fori_loop(0, 8, body, 0)

    def write(t, ob, wsem):
        s = t // CPB
        col = b0 + (t % CPB) * SUB
        return pltpu.async_copy(
            ob, out_hbm.at[pl.ds(s * DIM, DIM), pl.ds(col, SUB)], wsem)

    compute_idx(0, i0)
    fire(g0, i0, gsem0)
    compute_idx(1, i1)
    fire(g1, i1, gsem1)

    drain(g0, gsem0)
    compute_idx(2, i0)
    extract(0, g0, o0)
    fire(g0, i0, gsem0)
    write(0, o0, wsem0)
    drain(g1, gsem1)
    compute_idx(3, i1)
    extract(1, g1, o1)
    fire(g1, i1, gsem1)
    write(1, o1, wsem1)

    def pair_body(i, carry):
        a = 2 * i
        drain(g0, gsem0)
        compute_idx(a + 2, i0)
        pltpu.make_async_copy(o0, out_hbm.at[pl.ds(0, DIM), pl.ds(0, SUB)], wsem0).wait()
        extract(a, g0, o0)
        fire(g0, i0, gsem0)
        write(a, o0, wsem0)
        drain(g1, gsem1)
        compute_idx(a + 3, i1)
        pltpu.make_async_copy(o1, out_hbm.at[pl.ds(0, DIM), pl.ds(0, SUB)], wsem1).wait()
        extract(a + 1, g1, o1)
        fire(g1, i1, gsem1)
        write(a + 1, o1, wsem1)
        return carry

    lax.fori_loop(1, NCHUNK // 2 - 1, pair_body, 0)

    drain(g0, gsem0)
    pltpu.make_async_copy(o0, out_hbm.at[pl.ds(0, DIM), pl.ds(0, SUB)], wsem0).wait()
    extract(NCHUNK - 2, g0, o0)
    w0 = write(NCHUNK - 2, o0, wsem0)
    drain(g1, gsem1)
    pltpu.make_async_copy(o1, out_hbm.at[pl.ds(0, DIM), pl.ds(0, SUB)], wsem1).wait()
    extract(NCHUNK - 1, g1, o1)
    w1 = write(NCHUNK - 1, o1, wsem1)
    w0.wait()
    w1.wait()


def kernel(x, weight):
    w2 = jnp.pad(weight, ((0, 0), (0, DIM)))
    xt = x.T.astype(jnp.int32)
    out = _emb_kernel(w2, xt)
    return jnp.transpose(out.reshape(SEQ, DIM, BATCH), (2, 0, 1))
